# baseline (speedup 1.0000x reference)
import functools

import jax
import jax.numpy as jnp
from jax import lax
from jax.experimental import pallas as pl
from jax.experimental.pallas import tpu as pltpu
from jax.experimental.pallas import tpu_sc as plsc

_B, _H, _D = 16384, 50, 64
_TOTAL = _B * _H
_NC, _NS = 2, 16
_NW = _NC * _NS
_PER_W = _TOTAL // _NW
_CHUNK = 512
_GPB = 1
_ROWS = _CHUNK * _GPB
_NB = 2
_STEPS = _PER_W // _ROWS


def _sc_body(idx_hbm, table_hbm, out_hbm, idx_v, rows_v,
             gsem0, gsem1, osem0, osem1):
    gsems = (gsem0, gsem1)
    osems = (osem0, osem1)
    wid = lax.axis_index("s") * _NC + lax.axis_index("c")
    out_base = wid * _PER_W

    pltpu.sync_copy(idx_hbm.at[wid], idx_v)

    def issue_gathers(step, b):
        for j in range(_GPB):
            pltpu.async_copy(
                table_hbm.at[idx_v.at[step * _GPB + j]],
                rows_v.at[b, pl.ds(j * _CHUNK, _CHUNK)],
                gsems[b],
            )

    def wait_gathers(b):
        pltpu.make_async_copy(
            out_hbm.at[pl.ds(0, _ROWS)], rows_v.at[b], gsems[b]
        ).wait()

    def write_out(step, b):
        return pltpu.async_copy(
            rows_v.at[b],
            out_hbm.at[pl.ds(out_base + step * _ROWS, _ROWS)],
            osems[b],
        )

    for b in range(_NB):
        issue_gathers(b, b)

    def outer(gi, carry):
        for b in range(_NB):
            s = gi * _NB + b
            wait_gathers(b)
            out_copy = write_out(s, b)
            out_copy.wait()
            issue_gathers(s + _NB, b)
        return carry

    lax.fori_loop(0, _STEPS // _NB - 1, outer, 0, unroll=False)

    for b in range(_NB):
        s = _STEPS - _NB + b
        wait_gathers(b)
        write_out(s, b).wait()


def kernel(inp, table):
    idx = inp.reshape(_TOTAL).astype(jnp.int32)
    idx3 = idx.reshape(_NW, _STEPS * _GPB, _CHUNK)
    mesh = plsc.VectorSubcoreMesh(core_axis_name="c", subcore_axis_name="s")
    out = pl.kernel(
        _sc_body,
        out_type=jax.ShapeDtypeStruct((_TOTAL, _D), jnp.float32),
        mesh=mesh,
        compiler_params=pltpu.CompilerParams(use_tc_tiling_on_sc=False),
        scratch_types=[
            pltpu.VMEM((_STEPS * _GPB, _CHUNK), jnp.int32),
            pltpu.VMEM((_NB, _ROWS, _D), jnp.float32),
            pltpu.SemaphoreType.DMA,
            pltpu.SemaphoreType.DMA,
            pltpu.SemaphoreType.DMA,
            pltpu.SemaphoreType.DMA,
        ],
    )(idx3, table)
    return out.reshape(_B, _H, _D)


# speedup vs baseline: 1.1748x; 1.1748x over previous
import functools

import jax
import jax.numpy as jnp
from jax import lax
from jax.experimental import pallas as pl
from jax.experimental.pallas import tpu as pltpu
from jax.experimental.pallas import tpu_sc as plsc

_B, _H, _D = 16384, 50, 64
_TOTAL = _B * _H
_NC, _NS = 2, 16
_NW = _NC * _NS
_PER_W = _TOTAL // _NW
_CHUNK = 512
_GPB = 1
_ROWS = _CHUNK * _GPB
_NB = 2
_STEPS = _PER_W // _ROWS


def _sc_body(idx_hbm, table_hbm, out_hbm, idx_v, rows_v,
             gsem0, gsem1, osem0, osem1):
    gsems = (gsem0, gsem1)
    osems = (osem0, osem1)
    wid = lax.axis_index("s") * _NC + lax.axis_index("c")
    out_base = wid * _PER_W

    pltpu.sync_copy(idx_hbm.at[wid], idx_v)

    def issue_gathers(step, b):
        for j in range(_GPB):
            pltpu.async_copy(
                table_hbm.at[idx_v.at[step * _GPB + j]],
                rows_v.at[b, pl.ds(j * _CHUNK, _CHUNK)],
                gsems[b],
            )

    def wait_gathers(b):
        pltpu.make_async_copy(
            out_hbm.at[pl.ds(0, _ROWS)], rows_v.at[b], gsems[b]
        ).wait()

    def write_out(step, b):
        return pltpu.async_copy(
            rows_v.at[b],
            out_hbm.at[pl.ds(out_base + step * _ROWS, _ROWS)],
            osems[b],
        )

    for b in range(_NB):
        issue_gathers(b, b)

    def outer(gi, carry):
        for b in range(_NB):
            s = gi * _NB + b
            wait_gathers(b)
            out_copy = write_out(s, b)
            out_copy.wait()
            issue_gathers(s + _NB, b)
        return carry

    lax.fori_loop(0, _STEPS // _NB - 1, outer, 0, unroll=False)

    for b in range(_NB):
        s = _STEPS - _NB + b
        wait_gathers(b)
        write_out(s, b).wait()


_CB = 8192


def _tc_transpose_body(x_ref, o_ref):
    xt = x_ref[...].T.reshape(_CB // 2, 2, 64)
    o_ref[...] = jnp.concatenate([xt[:, 0, :], xt[:, 1, :]], axis=1)


def _linearize_table(table):
    table_t = table.T
    grid = (1000000 + _CB - 1) // _CB
    lin = pl.pallas_call(
        _tc_transpose_body,
        grid=(grid,),
        in_specs=[pl.BlockSpec((64, _CB), lambda i: (0, i))],
        out_specs=pl.BlockSpec((_CB // 2, 128), lambda i: (i, 0)),
        out_shape=jax.ShapeDtypeStruct((500000, 128), jnp.float32),
    )(table_t)
    return lin.reshape(1000000, 64)


def kernel(inp, table):
    table_lin = _linearize_table(table)
    idx = inp.reshape(_TOTAL).astype(jnp.int32)
    idx3 = idx.reshape(_NW, _STEPS * _GPB, _CHUNK)
    mesh = plsc.VectorSubcoreMesh(core_axis_name="c", subcore_axis_name="s")
    out = pl.kernel(
        _sc_body,
        out_type=jax.ShapeDtypeStruct((_TOTAL, _D), jnp.float32),
        mesh=mesh,
        compiler_params=pltpu.CompilerParams(use_tc_tiling_on_sc=False),
        scratch_types=[
            pltpu.VMEM((_STEPS * _GPB, _CHUNK), jnp.int32),
            pltpu.VMEM((_NB, _ROWS, _D), jnp.float32),
            pltpu.SemaphoreType.DMA,
            pltpu.SemaphoreType.DMA,
            pltpu.SemaphoreType.DMA,
            pltpu.SemaphoreType.DMA,
        ],
    )(idx3, table_lin)
    return out.reshape(_B, _H, _D)
